# baseline (speedup 1.0000x reference)
import functools

import jax
import jax.numpy as jnp
from jax import lax
from jax.experimental import pallas as pl
from jax.experimental.pallas import tpu as pltpu
from jax.experimental.pallas import tpu_sc as plsc

NC = 2
NS = 16
NW = NC * NS
SUB = 128
MACRO = 256
RW = 128
HW = 64
XL = 64
DEGL = 67


def _silu(v):
    return v * jax.nn.sigmoid(v)


def _sc_mesh():
    return plsc.VectorSubcoreMesh(
        core_axis_name="c", subcore_axis_name="s", num_cores=NC, num_subcores=NS
    )


def _sc_gather(ts, td, sidx3, didx3):
    ep = sidx3.shape[0] * MACRO
    ept = ep // NS
    nm = ept // MACRO
    nsub = MACRO // SUB
    nu = ept // SUB
    ich = next(d for d in (8, 5, 4, 2, 1) if nm % d == 0)
    upc = ich * nsub
    nch = nm // ich
    n_pad = ts.shape[0]
    rpt = n_pad // NS
    dt = ts.dtype

    @functools.partial(
        pl.kernel,
        out_type=(
            jax.ShapeDtypeStruct((ep, RW), dt),
            jax.ShapeDtypeStruct((ep, RW), dt),
        ),
        mesh=_sc_mesh(),
        scratch_types=(
            pltpu.VMEM((ich, nsub, SUB), jnp.int32),
            pltpu.VMEM((SUB, RW), dt),
            pltpu.VMEM((SUB, RW), dt),
            pltpu.VMEM_SHARED((n_pad, RW), dt),
            pltpu.SemaphoreType.DMA,
            pltpu.SemaphoreType.DMA,
            pltpu.SemaphoreType.DMA,
            pltpu.SemaphoreType.DMA,
        ),
    )
    def kfn(ts_hbm, td_hbm, si_hbm, di_hbm, gs_hbm, gd_hbm,
            idx_v, buf0, buf1, tab_s, gs0, gs1, ws0, ws1):
        cid = lax.axis_index("c")
        sid = lax.axis_index("s")
        r0 = sid * rpt
        base = sid * ept
        bufs = (buf0, buf1)
        gsem = (gs0, gs1)
        wsem = (ws0, ws1)

        def side(tab_hbm, i_hbm, out_hbm):
            pltpu.sync_copy(tab_hbm.at[pl.ds(r0, rpt)],
                            tab_s.at[pl.ds(r0, rpt)])
            pltpu.sync_copy(i_hbm.at[pl.ds(sid * nm, ich)], idx_v)
            plsc.subcore_barrier()

            def gather(mi, s, b):
                pltpu.async_copy(tab_s.at[idx_v.at[mi, s]], bufs[b],
                                 gsem[b])

            def wait_gather(b):
                pltpu.make_async_copy(tab_s.at[idx_v.at[0, 0]], bufs[b],
                                      gsem[b]).wait()

            def write(u, b):
                pltpu.async_copy(bufs[b], out_hbm.at[pl.ds(base + u * SUB,
                                                           SUB)], wsem[b])

            def wait_write(b):
                pltpu.make_async_copy(bufs[b], out_hbm.at[pl.ds(0, SUB)],
                                      wsem[b]).wait()

            gather(0, 0, 0)

            def chunk(c, carry):
                u0 = c * upc
                for ui in range(upc):
                    b = ui % 2
                    u = u0 + ui
                    if ui + 1 < upc:
                        @pl.when(u >= 1)
                        def _():
                            wait_write(1 - b)

                        gather((ui + 1) // nsub, (ui + 1) % nsub, 1 - b)
                        wait_gather(b)
                        write(u, b)
                    else:
                        @pl.when(u >= 1)
                        def _():
                            wait_write(1 - b)

                        wait_gather(b)
                        write(u, b)

                        @pl.when(c + 1 < nch)
                        def _():
                            pltpu.sync_copy(
                                i_hbm.at[pl.ds(sid * nm + (c + 1) * ich,
                                               ich)], idx_v)
                            gather(0, 0, 1 - b)
                return carry

            lax.fori_loop(0, nch, chunk, 0)
            wait_write((nu - 1) % 2)

        @pl.when(cid == 0)
        def _():
            side(ts_hbm, si_hbm, gs_hbm)

        @pl.when(cid == 1)
        def _():
            side(td_hbm, di_hbm, gd_hbm)

    return kfn(ts, td, sidx3, didx3)


def _sc_scatter(msg, didx3, n_pad):
    ep = didx3.shape[0] * MACRO
    epw = ep // NW
    nm = epw // MACRO
    nsub = MACRO // SUB
    nu = epw // SUB
    ich = next(d for d in (8, 5, 4, 2, 1) if nm % d == 0)
    upc = ich * nsub
    nch = nm // ich
    rpt = n_pad // NS
    f32 = jnp.float32
    zeros = jnp.zeros((n_pad, RW), f32)

    @functools.partial(
        pl.kernel,
        out_type=jax.ShapeDtypeStruct((NC, n_pad, RW), f32),
        mesh=_sc_mesh(),
        scratch_types=(
            pltpu.VMEM((ich, nsub, SUB), jnp.int32),
            pltpu.VMEM((SUB, RW), f32),
            pltpu.VMEM((SUB, RW), f32),
            pltpu.VMEM_SHARED((n_pad, RW), f32),
            pltpu.SemaphoreType.DMA,
            pltpu.SemaphoreType.DMA,
        ),
    )
    def kfn(msg_hbm, di_hbm, z_hbm, out_hbm, idx_v, buf0, buf1, acc_s,
            ls0, ls1):
        cid = lax.axis_index("c")
        sid = lax.axis_index("s")
        wid = cid * NS + sid
        r0 = sid * rpt
        base = wid * epw
        bufs = (buf0, buf1)
        lsem = (ls0, ls1)
        pltpu.sync_copy(z_hbm.at[pl.ds(r0, rpt)], acc_s.at[pl.ds(r0, rpt)])
        pltpu.sync_copy(di_hbm.at[pl.ds(wid * nm, ich)], idx_v)
        plsc.subcore_barrier()

        def load(u, b):
            pltpu.async_copy(msg_hbm.at[pl.ds(base + u * SUB, SUB)],
                             bufs[b], lsem[b])

        def wait_load(b):
            pltpu.make_async_copy(msg_hbm.at[pl.ds(0, SUB)], bufs[b],
                                  lsem[b]).wait()

        load(0, 0)
        load(1, 1)

        def chunk(c, carry):
            u0 = c * upc
            for ui in range(upc):
                b = ui % 2
                u = u0 + ui
                wait_load(b)
                pltpu.sync_copy(bufs[b],
                                acc_s.at[idx_v.at[ui // nsub, ui % nsub]],
                                add=True)

                @pl.when(u + 2 < nu)
                def _():
                    load(u + 2, b)

            @pl.when(c + 1 < nch)
            def _():
                pltpu.sync_copy(
                    di_hbm.at[pl.ds(wid * nm + (c + 1) * ich, ich)], idx_v)
            return carry

        lax.fori_loop(0, nch, chunk, 0)
        plsc.subcore_barrier()
        pltpu.sync_copy(acc_s.at[pl.ds(r0, rpt)],
                        out_hbm.at[cid, pl.ds(r0, rpt)])

    return kfn(msg, didx3, zeros)


def _tc_tables(h, x128, wa, wb):
    n = h.shape[0]
    f32 = jnp.float32

    def body(h_r, x_r, wa_r, wb_r, st_r, ts_r, td_r):
        hv = h_r[...]
        xv = x_r[...]
        st = st_r[...]
        p = jnp.dot(hv, wa_r[...], preferred_element_type=f32)
        q = jnp.dot(hv, wb_r[...], preferred_element_type=f32)
        ts_r[...] = jnp.dot(p, st, preferred_element_type=f32) + xv
        td_r[...] = jnp.dot(q, st, preferred_element_type=f32) + xv

    st = jnp.eye(HW, RW, dtype=f32)
    return pl.pallas_call(
        body,
        out_shape=(jax.ShapeDtypeStruct((n, RW), f32),
                   jax.ShapeDtypeStruct((n, RW), f32)),
    )(h, x128, wa, wb, st)


def _tc_edge_mlp(gs, gd, eattr, wr, we, b1, w2, b2, cw1, cb1, cw2,
                 e_real, row0, with_coord):
    ep = gs.shape[0]
    blk = 4096
    grid = ep // blk
    f32 = jnp.float32

    def body(*refs):
        if with_coord:
            (gs_r, gd_r, ea_r, sel_r, selt_r, wr_r, we_r, b1_r, w2_r, b2_r,
             cw1_r, cb1_r, cw2_r, out_r) = refs
        else:
            (gs_r, gd_r, ea_r, sel_r, selt_r, wr_r, we_r, b1_r, w2_r, b2_r,
             out_r) = refs
        i = pl.program_id(0)
        gsv = gs_r[...].astype(f32)
        gdv = gd_r[...].astype(f32)
        sel = sel_r[...]
        lane = lax.broadcasted_iota(jnp.int32, (1, RW), 1)
        xmask = jnp.where((lane >= XL) & (lane < XL + 3), 1.0, 0.0).astype(f32)
        xd = (gsv - gdv) * xmask
        radial = jnp.sum(xd * xd, axis=1, keepdims=True)
        pre = (jnp.dot(gsv + gdv, sel, preferred_element_type=f32)
               + radial * wr_r[...]
               + jnp.dot(ea_r[...], we_r[...], preferred_element_type=f32)
               + b1_r[...])
        u = jnp.dot(_silu(pre), w2_r[...], preferred_element_type=f32) + b2_r[...]
        mh = _silu(u)
        rows = row0 + i * blk + lax.broadcasted_iota(jnp.int32, (blk, 1), 0)
        emask = rows < e_real
        selt = selt_r[...]
        out = jnp.dot(mh, selt, preferred_element_type=f32)
        if with_coord:
            coef = jnp.dot(
                _silu(jnp.dot(mh, cw1_r[...], preferred_element_type=f32)
                      + cb1_r[...]),
                cw2_r[...], preferred_element_type=f32)
            inv = 1.0 / (jnp.sqrt(radial) + 1e-30)
            deghot = jnp.where(lane == DEGL, 1.0, 0.0).astype(f32)
            out = out + coef * (xd * inv) + deghot
        out_r[...] = jnp.where(emask, out, 0.0)

    def eb(s):
        return pl.BlockSpec((blk, s), lambda i: (i, 0))

    def wbs(shape):
        return pl.BlockSpec(shape, lambda i: tuple(0 for _ in shape))

    sel = jnp.eye(RW, HW, dtype=f32)
    selt = jnp.eye(HW, RW, dtype=f32)
    in_specs = [eb(RW), eb(RW), eb(eattr.shape[1]), wbs(sel.shape),
                wbs(selt.shape), wbs(wr.shape), wbs(we.shape), wbs(b1.shape),
                wbs(w2.shape), wbs(b2.shape)]
    args = [gs, gd, eattr, sel, selt, wr, we, b1, w2, b2]
    if with_coord:
        in_specs += [wbs(cw1.shape), wbs(cb1.shape), wbs(cw2.shape)]
        args += [cw1, cb1, cw2]

    return pl.pallas_call(
        body,
        grid=(grid,),
        in_specs=in_specs,
        out_specs=eb(RW),
        out_shape=jax.ShapeDtypeStruct((ep, RW), f32),
    )(*args)


def _tc_node1(h0, x128, parts, nw1a, nw1b, nb1, nw2, nb2, wa2, wb2):
    n = h0.shape[0]
    np_ = len(parts)
    f32 = jnp.float32

    def body(*refs):
        (h_r, x_r), ph_rs, (sel_r, selt_r, nw1a_r, nw1b_r, nb1_r,
                            nw2_r, nb2_r, wa2_r, wb2_r,
                            hr_r, ts2_r, td2_r) = (
            refs[:2], refs[2:2 + np_], refs[2 + np_:])
        agg = sum(p[0] + p[1] for p in ph_rs)
        sel = sel_r[...]
        selt = selt_r[...]
        lane = lax.broadcasted_iota(jnp.int32, (1, RW), 1)
        hn = jnp.dot(agg, sel, preferred_element_type=f32)
        degcol = jnp.where(
            lax.broadcasted_iota(jnp.int32, (RW, 1), 0) == DEGL,
            1.0, 0.0).astype(f32)
        deg = jnp.dot(agg, degcol, preferred_element_type=f32)
        xmask = jnp.where((lane >= XL) & (lane < XL + 3), 1.0, 0.0).astype(f32)
        xsum = agg * xmask
        xneigh = xsum / jnp.maximum(deg, 1.0)
        xout = x_r[...] + xneigh
        pre = (jnp.dot(h_r[...], nw1a_r[...], preferred_element_type=f32)
               + jnp.dot(hn, nw1b_r[...], preferred_element_type=f32)
               + nb1_r[...])
        h1 = jnp.dot(_silu(pre), nw2_r[...], preferred_element_type=f32) + nb2_r[...]
        hr = jnp.maximum(h1, 0.0)
        xr = jnp.maximum(xout, 0.0)
        hr_r[...] = hr
        p2 = jnp.dot(hr, wa2_r[...], preferred_element_type=f32)
        q2 = jnp.dot(hr, wb2_r[...], preferred_element_type=f32)
        ts2_r[...] = jnp.dot(p2, selt, preferred_element_type=f32) + xr
        td2_r[...] = jnp.dot(q2, selt, preferred_element_type=f32) + xr

    sel = jnp.eye(RW, HW, dtype=f32)
    selt = jnp.eye(HW, RW, dtype=f32)
    bn = n // 8

    def nb(s):
        return pl.BlockSpec((bn, s), lambda i: (i, 0))

    def pb():
        return pl.BlockSpec((2, bn, RW), lambda i: (0, i, 0))

    def wbs(shape):
        return pl.BlockSpec(shape, lambda i: tuple(0 for _ in shape))

    return pl.pallas_call(
        body,
        grid=(8,),
        in_specs=[nb(h0.shape[1]), nb(RW)] + [pb() for _ in parts]
        + [wbs(sel.shape), wbs(selt.shape), wbs(nw1a.shape),
           wbs(nw1b.shape), wbs(nb1.shape), wbs(nw2.shape), wbs(nb2.shape),
           wbs(wa2.shape), wbs(wb2.shape)],
        out_specs=(nb(HW), nb(RW), nb(RW)),
        out_shape=(jax.ShapeDtypeStruct((n, HW), f32),
                   jax.ShapeDtypeStruct((n, RW), f32),
                   jax.ShapeDtypeStruct((n, RW), f32)),
    )(h0, x128, *parts, sel, selt, nw1a, nw1b, nb1, nw2, nb2, wa2, wb2)


def _tc_node2(hr, parts, nw1a, nw1b, nb1, nw2, nb2, n_real):
    n_pad = hr.shape[0]
    np_ = len(parts)
    f32 = jnp.float32

    bn = n_pad // 8

    def body(*refs):
        (h_r,), ph_rs, (sel_r, nw1a_r, nw1b_r, nb1_r, nw2_r, nb2_r,
                        out_r) = refs[:1], refs[1:1 + np_], refs[1 + np_:]
        i = pl.program_id(0)
        agg = sum(p[0] + p[1] for p in ph_rs)
        hn = jnp.dot(agg, sel_r[...], preferred_element_type=f32)
        pre = (jnp.dot(h_r[...], nw1a_r[...], preferred_element_type=f32)
               + jnp.dot(hn, nw1b_r[...], preferred_element_type=f32)
               + nb1_r[...])
        h2 = jnp.dot(_silu(pre), nw2_r[...], preferred_element_type=f32) + nb2_r[...]
        rows = i * bn + lax.broadcasted_iota(jnp.int32, (bn, 1), 0)
        h2 = jnp.where(rows < n_real, h2, 0.0)
        part = jnp.sum(h2, axis=0, keepdims=True)

        @pl.when(i == 0)
        def _():
            out_r[...] = part

        @pl.when(i > 0)
        def _():
            out_r[...] = out_r[...] + part

    sel = jnp.eye(RW, HW, dtype=f32)

    def nb(s):
        return pl.BlockSpec((bn, s), lambda i: (i, 0))

    def pb():
        return pl.BlockSpec((2, bn, RW), lambda i: (0, i, 0))

    def wbs(shape):
        return pl.BlockSpec(shape, lambda i: tuple(0 for _ in shape))

    return pl.pallas_call(
        body,
        grid=(8,),
        in_specs=[nb(HW)] + [pb() for _ in parts]
        + [wbs(sel.shape), wbs(nw1a.shape), wbs(nw1b.shape),
           wbs(nb1.shape), wbs(nw2.shape), wbs(nb2.shape)],
        out_specs=pl.BlockSpec((1, HW), lambda i: (0, 0)),
        out_shape=jax.ShapeDtypeStruct((1, HW), f32),
    )(hr, *parts, sel, nw1a, nw1b, nb1, nw2, nb2)


def kernel(node_feat, coord_feat, edge_index, edge_attr,
           c1_e_w1, c1_e_b1, c1_e_w2, c1_e_b2, c1_n_w1, c1_n_b1, c1_n_w2,
           c1_n_b2, c1_c_w1, c1_c_b1, c1_c_w2,
           c2_e_w1, c2_e_b1, c2_e_w2, c2_e_b2, c2_n_w1, c2_n_b1, c2_n_w2,
           c2_n_b2, c2_c_w1, c2_c_b1, c2_c_w2):
    n, f_in = node_feat.shape
    e = edge_index.shape[1]
    hw = c1_e_w2.shape[0]
    c = coord_feat.shape[1]

    unit = NW * MACRO
    e_pad = ((e + unit - 1) // unit) * unit
    nunit = 16 * NS
    n_pad = ((n + nunit - 1) // nunit) * nunit

    src = jnp.pad(edge_index[0], (0, e_pad - e))
    dst = jnp.pad(edge_index[1], (0, e_pad - e))
    sidx3 = src.reshape(e_pad // MACRO, MACRO // SUB, SUB)
    didx3 = dst.reshape(e_pad // MACRO, MACRO // SUB, SUB)
    eattr = jnp.pad(edge_attr, ((0, e_pad - e), (0, 0)))
    node_feat = jnp.pad(node_feat, ((0, n_pad - n), (0, 0)))
    x128 = jnp.pad(coord_feat, ((0, n_pad - n), (XL, RW - XL - c)))

    wa1 = c1_e_w1[:f_in]
    wb1 = c1_e_w1[f_in:2 * f_in]
    wr1 = c1_e_w1[2 * f_in:2 * f_in + 1]
    we1 = c1_e_w1[2 * f_in + 1:]
    b1e = c1_e_b1.reshape(1, hw)
    b2e = c1_e_b2.reshape(1, hw)
    cb1 = c1_c_b1.reshape(1, hw)
    nw1a = c1_n_w1[:f_in]
    nw1b = c1_n_w1[f_in:]
    nb1 = c1_n_b1.reshape(1, hw)
    nb2 = c1_n_b2.reshape(1, hw)
    wa2 = c2_e_w1[:hw]
    wb2 = c2_e_w1[hw:2 * hw]
    wr2 = c2_e_w1[2 * hw:2 * hw + 1]
    we2 = c2_e_w1[2 * hw + 1:]
    b1e2 = c2_e_b1.reshape(1, hw)
    b2e2 = c2_e_b2.reshape(1, hw)
    nw1a2 = c2_n_w1[:hw]
    nw1b2 = c2_n_w1[hw:]
    nb12 = c2_n_b1.reshape(1, hw)
    nb22 = c2_n_b2.reshape(1, hw)

    nstream = 4
    nmac = e_pad // MACRO
    halves = []
    for hidx in range(nstream):
        mac0 = hidx * (nmac // nstream)
        row0 = mac0 * MACRO
        halves.append((
            row0,
            sidx3[mac0:mac0 + nmac // nstream],
            didx3[mac0:mac0 + nmac // nstream],
            eattr[row0:row0 + e_pad // nstream],
        ))

    def conv_edges(ts, td, ew, wr, we, b1, b2, cw1, cb1, cw2, with_coord):
        parts = []
        for row0, sa, da, ea in halves:
            gsx, gdx = _sc_gather(ts, td, sa, da)
            msgx = _tc_edge_mlp(gsx, gdx, ea, wr, we, b1, ew, b2,
                                cw1, cb1, cw2, e, row0, with_coord)
            parts.append(_sc_scatter(msgx, da, n_pad))
        return parts

    ts1, td1 = _tc_tables(node_feat, x128, wa1, wb1)
    ph1 = conv_edges(ts1, td1, c1_e_w2, wr1, we1, b1e, b2e,
                     c1_c_w1, cb1, c1_c_w2, True)
    hr, ts2, td2 = _tc_node1(node_feat, x128, ph1, nw1a, nw1b, nb1,
                             c1_n_w2, nb2, wa2, wb2)

    ph2 = conv_edges(ts2, td2, c2_e_w2, wr2, we2, b1e2, b2e2,
                     None, None, None, False)
    return _tc_node2(hr, ph2, nw1a2, nw1b2, nb12, c2_n_w2, nb22, n)


# speedup vs baseline: 1.0378x; 1.0378x over previous
import functools

import jax
import jax.numpy as jnp
from jax import lax
from jax.experimental import pallas as pl
from jax.experimental.pallas import tpu as pltpu
from jax.experimental.pallas import tpu_sc as plsc

NC = 2
NS = 16
NW = NC * NS
SUB = 128
MACRO = 256
RW = 128
HW = 64
XL = 64
DEGL = 67


def _silu(v):
    return v * jax.nn.sigmoid(v)


def _sc_mesh():
    return plsc.VectorSubcoreMesh(
        core_axis_name="c", subcore_axis_name="s", num_cores=NC, num_subcores=NS
    )


def _sc_gather(ts, td, sidx3, didx3):
    ep = sidx3.shape[0] * MACRO
    ept = ep // NS
    nm = ept // MACRO
    nsub = MACRO // SUB
    nu = ept // SUB
    ich = next(d for d in (8, 5, 4, 2, 1) if nm % d == 0)
    upc = ich * nsub
    nch = nm // ich
    n_pad = ts.shape[0]
    rpt = n_pad // NS
    dt = ts.dtype

    @functools.partial(
        pl.kernel,
        out_type=(
            jax.ShapeDtypeStruct((ep, RW), dt),
            jax.ShapeDtypeStruct((ep, RW), dt),
        ),
        mesh=_sc_mesh(),
        scratch_types=(
            pltpu.VMEM((ich, nsub, SUB), jnp.int32),
            pltpu.VMEM((SUB, RW), dt),
            pltpu.VMEM((SUB, RW), dt),
            pltpu.VMEM_SHARED((n_pad, RW), dt),
            pltpu.SemaphoreType.DMA,
            pltpu.SemaphoreType.DMA,
            pltpu.SemaphoreType.DMA,
            pltpu.SemaphoreType.DMA,
        ),
    )
    def kfn(ts_hbm, td_hbm, si_hbm, di_hbm, gs_hbm, gd_hbm,
            idx_v, buf0, buf1, tab_s, gs0, gs1, ws0, ws1):
        cid = lax.axis_index("c")
        sid = lax.axis_index("s")
        r0 = sid * rpt
        base = sid * ept
        bufs = (buf0, buf1)
        gsem = (gs0, gs1)
        wsem = (ws0, ws1)

        def side(tab_hbm, i_hbm, out_hbm):
            pltpu.sync_copy(tab_hbm.at[pl.ds(r0, rpt)],
                            tab_s.at[pl.ds(r0, rpt)])
            pltpu.sync_copy(i_hbm.at[pl.ds(sid * nm, ich)], idx_v)
            plsc.subcore_barrier()

            def gather(mi, s, b):
                pltpu.async_copy(tab_s.at[idx_v.at[mi, s]], bufs[b],
                                 gsem[b])

            def wait_gather(b):
                pltpu.make_async_copy(tab_s.at[idx_v.at[0, 0]], bufs[b],
                                      gsem[b]).wait()

            def write(u, b):
                pltpu.async_copy(bufs[b], out_hbm.at[pl.ds(base + u * SUB,
                                                           SUB)], wsem[b])

            def wait_write(b):
                pltpu.make_async_copy(bufs[b], out_hbm.at[pl.ds(0, SUB)],
                                      wsem[b]).wait()

            gather(0, 0, 0)

            def chunk(c, carry):
                u0 = c * upc
                for ui in range(upc):
                    b = ui % 2
                    u = u0 + ui
                    if ui + 1 < upc:
                        @pl.when(u >= 1)
                        def _():
                            wait_write(1 - b)

                        gather((ui + 1) // nsub, (ui + 1) % nsub, 1 - b)
                        wait_gather(b)
                        write(u, b)
                    else:
                        @pl.when(u >= 1)
                        def _():
                            wait_write(1 - b)

                        wait_gather(b)
                        write(u, b)

                        @pl.when(c + 1 < nch)
                        def _():
                            pltpu.sync_copy(
                                i_hbm.at[pl.ds(sid * nm + (c + 1) * ich,
                                               ich)], idx_v)
                            gather(0, 0, 1 - b)
                return carry

            lax.fori_loop(0, nch, chunk, 0)
            wait_write((nu - 1) % 2)

        @pl.when(cid == 0)
        def _():
            side(ts_hbm, si_hbm, gs_hbm)

        @pl.when(cid == 1)
        def _():
            side(td_hbm, di_hbm, gd_hbm)

    return kfn(ts, td, sidx3, didx3)


def _sc_scatter(msg, didx3, n_pad):
    ep = didx3.shape[0] * MACRO
    epw = ep // NW
    nm = epw // MACRO
    nsub = MACRO // SUB
    nu = epw // SUB
    ich = next(d for d in (8, 5, 4, 2, 1) if nm % d == 0)
    upc = ich * nsub
    nch = nm // ich
    rpt = n_pad // NS
    f32 = jnp.float32
    zeros = jnp.zeros((n_pad, RW), f32)

    @functools.partial(
        pl.kernel,
        out_type=jax.ShapeDtypeStruct((NC, n_pad, RW), f32),
        mesh=_sc_mesh(),
        scratch_types=(
            pltpu.VMEM((ich, nsub, SUB), jnp.int32),
            pltpu.VMEM((SUB, RW), f32),
            pltpu.VMEM((SUB, RW), f32),
            pltpu.VMEM_SHARED((n_pad, RW), f32),
            pltpu.SemaphoreType.DMA,
            pltpu.SemaphoreType.DMA,
        ),
    )
    def kfn(msg_hbm, di_hbm, z_hbm, out_hbm, idx_v, buf0, buf1, acc_s,
            ls0, ls1):
        cid = lax.axis_index("c")
        sid = lax.axis_index("s")
        wid = cid * NS + sid
        r0 = sid * rpt
        base = wid * epw
        bufs = (buf0, buf1)
        lsem = (ls0, ls1)
        pltpu.sync_copy(z_hbm.at[pl.ds(r0, rpt)], acc_s.at[pl.ds(r0, rpt)])
        pltpu.sync_copy(di_hbm.at[pl.ds(wid * nm, ich)], idx_v)
        plsc.subcore_barrier()

        def load(u, b):
            pltpu.async_copy(msg_hbm.at[pl.ds(base + u * SUB, SUB)],
                             bufs[b], lsem[b])

        def wait_load(b):
            pltpu.make_async_copy(msg_hbm.at[pl.ds(0, SUB)], bufs[b],
                                  lsem[b]).wait()

        load(0, 0)
        load(1, 1)

        def chunk(c, carry):
            u0 = c * upc
            for ui in range(upc):
                b = ui % 2
                u = u0 + ui
                wait_load(b)
                pltpu.sync_copy(bufs[b],
                                acc_s.at[idx_v.at[ui // nsub, ui % nsub]],
                                add=True)

                @pl.when(u + 2 < nu)
                def _():
                    load(u + 2, b)

            @pl.when(c + 1 < nch)
            def _():
                pltpu.sync_copy(
                    di_hbm.at[pl.ds(wid * nm + (c + 1) * ich, ich)], idx_v)
            return carry

        lax.fori_loop(0, nch, chunk, 0)
        plsc.subcore_barrier()
        pltpu.sync_copy(acc_s.at[pl.ds(r0, rpt)],
                        out_hbm.at[cid, pl.ds(r0, rpt)])

    return kfn(msg, didx3, zeros)


def _tc_tables(h, x128, wa, wb):
    n = h.shape[0]
    f32 = jnp.float32

    def body(h_r, x_r, wa_r, wb_r, st_r, ts_r, td_r):
        hv = h_r[...]
        xv = x_r[...]
        st = st_r[...]
        p = jnp.dot(hv, wa_r[...], preferred_element_type=f32)
        q = jnp.dot(hv, wb_r[...], preferred_element_type=f32)
        ts_r[...] = jnp.dot(p, st, preferred_element_type=f32) + xv
        td_r[...] = jnp.dot(q, st, preferred_element_type=f32) + xv

    st = jnp.eye(HW, RW, dtype=f32)
    return pl.pallas_call(
        body,
        out_shape=(jax.ShapeDtypeStruct((n, RW), f32),
                   jax.ShapeDtypeStruct((n, RW), f32)),
    )(h, x128, wa, wb, st)


def _tc_edge_mlp(gs, gd, eattr, wr, we, b1, w2, b2, cw1, cb1, cw2,
                 e_real, row0, with_coord):
    ep = gs.shape[0]
    blk = 4096
    grid = ep // blk
    f32 = jnp.float32

    def body(*refs):
        if with_coord:
            (gs_r, gd_r, ea_r, sel_r, selt_r, wr_r, we_r, b1_r, w2_r, b2_r,
             cw1_r, cb1_r, cw2_r, out_r) = refs
        else:
            (gs_r, gd_r, ea_r, sel_r, selt_r, wr_r, we_r, b1_r, w2_r, b2_r,
             out_r) = refs
        i = pl.program_id(0)
        gsv = gs_r[...].astype(f32)
        gdv = gd_r[...].astype(f32)
        sel = sel_r[...]
        lane = lax.broadcasted_iota(jnp.int32, (1, RW), 1)
        xmask = jnp.where((lane >= XL) & (lane < XL + 3), 1.0, 0.0).astype(f32)
        xd = (gsv - gdv) * xmask
        radial = jnp.sum(xd * xd, axis=1, keepdims=True)
        pre = (jnp.dot(gsv + gdv, sel, preferred_element_type=f32)
               + radial * wr_r[...]
               + jnp.dot(ea_r[...], we_r[...], preferred_element_type=f32)
               + b1_r[...])
        u = jnp.dot(_silu(pre), w2_r[...], preferred_element_type=f32) + b2_r[...]
        mh = _silu(u)
        rows = row0 + i * blk + lax.broadcasted_iota(jnp.int32, (blk, 1), 0)
        emask = rows < e_real
        selt = selt_r[...]
        out = jnp.dot(mh, selt, preferred_element_type=f32)
        if with_coord:
            coef = jnp.dot(
                _silu(jnp.dot(mh, cw1_r[...], preferred_element_type=f32)
                      + cb1_r[...]),
                cw2_r[...], preferred_element_type=f32)
            inv = 1.0 / (jnp.sqrt(radial) + 1e-30)
            deghot = jnp.where(lane == DEGL, 1.0, 0.0).astype(f32)
            out = out + coef * (xd * inv) + deghot
        out_r[...] = jnp.where(emask, out, 0.0)

    def eb(s):
        return pl.BlockSpec((blk, s), lambda i: (i, 0))

    def wbs(shape):
        return pl.BlockSpec(shape, lambda i: tuple(0 for _ in shape))

    sel = jnp.eye(RW, HW, dtype=f32)
    selt = jnp.eye(HW, RW, dtype=f32)
    in_specs = [eb(RW), eb(RW), eb(eattr.shape[1]), wbs(sel.shape),
                wbs(selt.shape), wbs(wr.shape), wbs(we.shape), wbs(b1.shape),
                wbs(w2.shape), wbs(b2.shape)]
    args = [gs, gd, eattr, sel, selt, wr, we, b1, w2, b2]
    if with_coord:
        in_specs += [wbs(cw1.shape), wbs(cb1.shape), wbs(cw2.shape)]
        args += [cw1, cb1, cw2]

    return pl.pallas_call(
        body,
        grid=(grid,),
        in_specs=in_specs,
        out_specs=eb(RW),
        out_shape=jax.ShapeDtypeStruct((ep, RW), f32),
    )(*args)


def _tc_node1(h0, x128, parts, nw1a, nw1b, nb1, nw2, nb2, wa2, wb2):
    n = h0.shape[0]
    np_ = len(parts)
    f32 = jnp.float32

    def body(*refs):
        (h_r, x_r), ph_rs, (sel_r, selt_r, nw1a_r, nw1b_r, nb1_r,
                            nw2_r, nb2_r, wa2_r, wb2_r,
                            hr_r, ts2_r, td2_r) = (
            refs[:2], refs[2:2 + np_], refs[2 + np_:])
        agg = sum(p[0] + p[1] for p in ph_rs)
        sel = sel_r[...]
        selt = selt_r[...]
        lane = lax.broadcasted_iota(jnp.int32, (1, RW), 1)
        hn = jnp.dot(agg, sel, preferred_element_type=f32)
        degcol = jnp.where(
            lax.broadcasted_iota(jnp.int32, (RW, 1), 0) == DEGL,
            1.0, 0.0).astype(f32)
        deg = jnp.dot(agg, degcol, preferred_element_type=f32)
        xmask = jnp.where((lane >= XL) & (lane < XL + 3), 1.0, 0.0).astype(f32)
        xsum = agg * xmask
        xneigh = xsum / jnp.maximum(deg, 1.0)
        xout = x_r[...] + xneigh
        pre = (jnp.dot(h_r[...], nw1a_r[...], preferred_element_type=f32)
               + jnp.dot(hn, nw1b_r[...], preferred_element_type=f32)
               + nb1_r[...])
        h1 = jnp.dot(_silu(pre), nw2_r[...], preferred_element_type=f32) + nb2_r[...]
        hr = jnp.maximum(h1, 0.0)
        xr = jnp.maximum(xout, 0.0)
        hr_r[...] = hr
        p2 = jnp.dot(hr, wa2_r[...], preferred_element_type=f32)
        q2 = jnp.dot(hr, wb2_r[...], preferred_element_type=f32)
        ts2_r[...] = jnp.dot(p2, selt, preferred_element_type=f32) + xr
        td2_r[...] = jnp.dot(q2, selt, preferred_element_type=f32) + xr

    sel = jnp.eye(RW, HW, dtype=f32)
    selt = jnp.eye(HW, RW, dtype=f32)
    bn = n // 8

    def nb(s):
        return pl.BlockSpec((bn, s), lambda i: (i, 0))

    def pb():
        return pl.BlockSpec((2, bn, RW), lambda i: (0, i, 0))

    def wbs(shape):
        return pl.BlockSpec(shape, lambda i: tuple(0 for _ in shape))

    return pl.pallas_call(
        body,
        grid=(8,),
        in_specs=[nb(h0.shape[1]), nb(RW)] + [pb() for _ in parts]
        + [wbs(sel.shape), wbs(selt.shape), wbs(nw1a.shape),
           wbs(nw1b.shape), wbs(nb1.shape), wbs(nw2.shape), wbs(nb2.shape),
           wbs(wa2.shape), wbs(wb2.shape)],
        out_specs=(nb(HW), nb(RW), nb(RW)),
        out_shape=(jax.ShapeDtypeStruct((n, HW), f32),
                   jax.ShapeDtypeStruct((n, RW), f32),
                   jax.ShapeDtypeStruct((n, RW), f32)),
    )(h0, x128, *parts, sel, selt, nw1a, nw1b, nb1, nw2, nb2, wa2, wb2)


def _tc_node2(hr, parts, nw1a, nw1b, nb1, nw2, nb2, n_real):
    n_pad = hr.shape[0]
    np_ = len(parts)
    f32 = jnp.float32

    bn = n_pad // 8

    def body(*refs):
        (h_r,), ph_rs, (sel_r, nw1a_r, nw1b_r, nb1_r, nw2_r, nb2_r,
                        out_r) = refs[:1], refs[1:1 + np_], refs[1 + np_:]
        i = pl.program_id(0)
        agg = sum(p[0] + p[1] for p in ph_rs)
        hn = jnp.dot(agg, sel_r[...], preferred_element_type=f32)
        pre = (jnp.dot(h_r[...], nw1a_r[...], preferred_element_type=f32)
               + jnp.dot(hn, nw1b_r[...], preferred_element_type=f32)
               + nb1_r[...])
        h2 = jnp.dot(_silu(pre), nw2_r[...], preferred_element_type=f32) + nb2_r[...]
        rows = i * bn + lax.broadcasted_iota(jnp.int32, (bn, 1), 0)
        h2 = jnp.where(rows < n_real, h2, 0.0)
        part = jnp.sum(h2, axis=0, keepdims=True)

        @pl.when(i == 0)
        def _():
            out_r[...] = part

        @pl.when(i > 0)
        def _():
            out_r[...] = out_r[...] + part

    sel = jnp.eye(RW, HW, dtype=f32)

    def nb(s):
        return pl.BlockSpec((bn, s), lambda i: (i, 0))

    def pb():
        return pl.BlockSpec((2, bn, RW), lambda i: (0, i, 0))

    def wbs(shape):
        return pl.BlockSpec(shape, lambda i: tuple(0 for _ in shape))

    return pl.pallas_call(
        body,
        grid=(8,),
        in_specs=[nb(HW)] + [pb() for _ in parts]
        + [wbs(sel.shape), wbs(nw1a.shape), wbs(nw1b.shape),
           wbs(nb1.shape), wbs(nw2.shape), wbs(nb2.shape)],
        out_specs=pl.BlockSpec((1, HW), lambda i: (0, 0)),
        out_shape=jax.ShapeDtypeStruct((1, HW), f32),
    )(hr, *parts, sel, nw1a, nw1b, nb1, nw2, nb2)


def kernel(node_feat, coord_feat, edge_index, edge_attr,
           c1_e_w1, c1_e_b1, c1_e_w2, c1_e_b2, c1_n_w1, c1_n_b1, c1_n_w2,
           c1_n_b2, c1_c_w1, c1_c_b1, c1_c_w2,
           c2_e_w1, c2_e_b1, c2_e_w2, c2_e_b2, c2_n_w1, c2_n_b1, c2_n_w2,
           c2_n_b2, c2_c_w1, c2_c_b1, c2_c_w2):
    n, f_in = node_feat.shape
    e = edge_index.shape[1]
    hw = c1_e_w2.shape[0]
    c = coord_feat.shape[1]

    unit = NW * MACRO
    e_pad = ((e + unit - 1) // unit) * unit
    nunit = 16 * NS
    n_pad = ((n + nunit - 1) // nunit) * nunit

    src = jnp.pad(edge_index[0], (0, e_pad - e))
    dst = jnp.pad(edge_index[1], (0, e_pad - e))
    sidx3 = src.reshape(e_pad // MACRO, MACRO // SUB, SUB)
    didx3 = dst.reshape(e_pad // MACRO, MACRO // SUB, SUB)
    eattr = jnp.pad(edge_attr, ((0, e_pad - e), (0, 0)))
    node_feat = jnp.pad(node_feat, ((0, n_pad - n), (0, 0)))
    x128 = jnp.pad(coord_feat, ((0, n_pad - n), (XL, RW - XL - c)))

    wa1 = c1_e_w1[:f_in]
    wb1 = c1_e_w1[f_in:2 * f_in]
    wr1 = c1_e_w1[2 * f_in:2 * f_in + 1]
    we1 = c1_e_w1[2 * f_in + 1:]
    b1e = c1_e_b1.reshape(1, hw)
    b2e = c1_e_b2.reshape(1, hw)
    cb1 = c1_c_b1.reshape(1, hw)
    nw1a = c1_n_w1[:f_in]
    nw1b = c1_n_w1[f_in:]
    nb1 = c1_n_b1.reshape(1, hw)
    nb2 = c1_n_b2.reshape(1, hw)
    wa2 = c2_e_w1[:hw]
    wb2 = c2_e_w1[hw:2 * hw]
    wr2 = c2_e_w1[2 * hw:2 * hw + 1]
    we2 = c2_e_w1[2 * hw + 1:]
    b1e2 = c2_e_b1.reshape(1, hw)
    b2e2 = c2_e_b2.reshape(1, hw)
    nw1a2 = c2_n_w1[:hw]
    nw1b2 = c2_n_w1[hw:]
    nb12 = c2_n_b1.reshape(1, hw)
    nb22 = c2_n_b2.reshape(1, hw)

    nstream = 2
    nmac = e_pad // MACRO
    halves = []
    for hidx in range(nstream):
        mac0 = hidx * (nmac // nstream)
        row0 = mac0 * MACRO
        halves.append((
            row0,
            sidx3[mac0:mac0 + nmac // nstream],
            didx3[mac0:mac0 + nmac // nstream],
            eattr[row0:row0 + e_pad // nstream],
        ))

    def conv_edges(ts, td, ew, wr, we, b1, b2, cw1, cb1, cw2, with_coord):
        parts = []
        for row0, sa, da, ea in halves:
            gsx, gdx = _sc_gather(ts, td, sa, da)
            msgx = _tc_edge_mlp(gsx, gdx, ea, wr, we, b1, ew, b2,
                                cw1, cb1, cw2, e, row0, with_coord)
            parts.append(_sc_scatter(msgx, da, n_pad))
        return parts

    ts1, td1 = _tc_tables(node_feat, x128, wa1, wb1)
    ph1 = conv_edges(ts1, td1, c1_e_w2, wr1, we1, b1e, b2e,
                     c1_c_w1, cb1, c1_c_w2, True)
    hr, ts2, td2 = _tc_node1(node_feat, x128, ph1, nw1a, nw1b, nb1,
                             c1_n_w2, nb2, wa2, wb2)

    ph2 = conv_edges(ts2, td2, c2_e_w2, wr2, we2, b1e2, b2e2,
                     None, None, None, False)
    return _tc_node2(hr, ph2, nw1a2, nw1b2, nb12, c2_n_w2, nb22, n)


# speedup vs baseline: 1.0400x; 1.0021x over previous
import functools

import jax
import jax.numpy as jnp
from jax import lax
from jax.experimental import pallas as pl
from jax.experimental.pallas import tpu as pltpu
from jax.experimental.pallas import tpu_sc as plsc

NC = 2
NS = 16
NW = NC * NS
SUB = 128
MACRO = 256
RW = 128
HW = 64
XL = 64
DEGL = 67


def _silu(v):
    return v * jax.nn.sigmoid(v)


def _sc_mesh():
    return plsc.VectorSubcoreMesh(
        core_axis_name="c", subcore_axis_name="s", num_cores=NC, num_subcores=NS
    )


def _sc_gather(ts, td, sidx3, didx3):
    ep = sidx3.shape[0] * MACRO
    ept = ep // NS
    nm = ept // MACRO
    nsub = MACRO // SUB
    nu = ept // SUB
    ich = next(d for d in (8, 5, 4, 2, 1) if nm % d == 0)
    upc = ich * nsub
    nch = nm // ich
    n_pad = ts.shape[0]
    rpt = n_pad // NS
    dt = ts.dtype

    @functools.partial(
        pl.kernel,
        out_type=(
            jax.ShapeDtypeStruct((ep, RW), dt),
            jax.ShapeDtypeStruct((ep, RW), dt),
        ),
        mesh=_sc_mesh(),
        scratch_types=(
            pltpu.VMEM((ich, nsub, SUB), jnp.int32),
            pltpu.VMEM((SUB, RW), dt),
            pltpu.VMEM((SUB, RW), dt),
            pltpu.VMEM_SHARED((n_pad, RW), dt),
            pltpu.SemaphoreType.DMA,
            pltpu.SemaphoreType.DMA,
            pltpu.SemaphoreType.DMA,
            pltpu.SemaphoreType.DMA,
        ),
    )
    def kfn(ts_hbm, td_hbm, si_hbm, di_hbm, gs_hbm, gd_hbm,
            idx_v, buf0, buf1, tab_s, gs0, gs1, ws0, ws1):
        cid = lax.axis_index("c")
        sid = lax.axis_index("s")
        r0 = sid * rpt
        base = sid * ept
        bufs = (buf0, buf1)
        gsem = (gs0, gs1)
        wsem = (ws0, ws1)

        def side(tab_hbm, i_hbm, out_hbm):
            pltpu.sync_copy(tab_hbm.at[pl.ds(r0, rpt)],
                            tab_s.at[pl.ds(r0, rpt)])
            pltpu.sync_copy(i_hbm.at[pl.ds(sid * nm, ich)], idx_v)
            plsc.subcore_barrier()

            def gather(mi, s, b):
                pltpu.async_copy(tab_s.at[idx_v.at[mi, s]], bufs[b],
                                 gsem[b])

            def wait_gather(b):
                pltpu.make_async_copy(tab_s.at[idx_v.at[0, 0]], bufs[b],
                                      gsem[b]).wait()

            def write(u, b):
                pltpu.async_copy(bufs[b], out_hbm.at[pl.ds(base + u * SUB,
                                                           SUB)], wsem[b])

            def wait_write(b):
                pltpu.make_async_copy(bufs[b], out_hbm.at[pl.ds(0, SUB)],
                                      wsem[b]).wait()

            gather(0, 0, 0)

            def chunk(c, carry):
                u0 = c * upc
                for ui in range(upc):
                    b = ui % 2
                    u = u0 + ui
                    if ui + 1 < upc:
                        @pl.when(u >= 1)
                        def _():
                            wait_write(1 - b)

                        gather((ui + 1) // nsub, (ui + 1) % nsub, 1 - b)
                        wait_gather(b)
                        write(u, b)
                    else:
                        @pl.when(u >= 1)
                        def _():
                            wait_write(1 - b)

                        wait_gather(b)
                        write(u, b)

                        @pl.when(c + 1 < nch)
                        def _():
                            pltpu.sync_copy(
                                i_hbm.at[pl.ds(sid * nm + (c + 1) * ich,
                                               ich)], idx_v)
                            gather(0, 0, 1 - b)
                return carry

            lax.fori_loop(0, nch, chunk, 0)
            wait_write((nu - 1) % 2)

        @pl.when(cid == 0)
        def _():
            side(ts_hbm, si_hbm, gs_hbm)

        @pl.when(cid == 1)
        def _():
            side(td_hbm, di_hbm, gd_hbm)

    return kfn(ts, td, sidx3, didx3)


def _sc_scatter(msg, didx3, n_pad):
    ep = didx3.shape[0] * MACRO
    epw = ep // NW
    nm = epw // MACRO
    nsub = MACRO // SUB
    nu = epw // SUB
    ich = next(d for d in (8, 5, 4, 2, 1) if nm % d == 0)
    upc = ich * nsub
    nch = nm // ich
    rpt = n_pad // NS
    f32 = jnp.float32
    zeros = jnp.zeros((n_pad, RW), f32)

    @functools.partial(
        pl.kernel,
        out_type=jax.ShapeDtypeStruct((NC, n_pad, RW), f32),
        mesh=_sc_mesh(),
        scratch_types=(
            pltpu.VMEM((ich, nsub, SUB), jnp.int32),
            pltpu.VMEM((SUB, RW), f32),
            pltpu.VMEM((SUB, RW), f32),
            pltpu.VMEM_SHARED((n_pad, RW), f32),
            pltpu.SemaphoreType.DMA,
            pltpu.SemaphoreType.DMA,
        ),
    )
    def kfn(msg_hbm, di_hbm, z_hbm, out_hbm, idx_v, buf0, buf1, acc_s,
            ls0, ls1):
        cid = lax.axis_index("c")
        sid = lax.axis_index("s")
        wid = cid * NS + sid
        r0 = sid * rpt
        base = wid * epw
        bufs = (buf0, buf1)
        lsem = (ls0, ls1)
        pltpu.sync_copy(z_hbm.at[pl.ds(r0, rpt)], acc_s.at[pl.ds(r0, rpt)])
        pltpu.sync_copy(di_hbm.at[pl.ds(wid * nm, ich)], idx_v)
        plsc.subcore_barrier()

        def load(u, b):
            pltpu.async_copy(msg_hbm.at[pl.ds(base + u * SUB, SUB)],
                             bufs[b], lsem[b])

        def wait_load(b):
            pltpu.make_async_copy(msg_hbm.at[pl.ds(0, SUB)], bufs[b],
                                  lsem[b]).wait()

        load(0, 0)
        load(1, 1)

        def chunk(c, carry):
            u0 = c * upc
            for ui in range(upc):
                b = ui % 2
                u = u0 + ui
                wait_load(b)
                pltpu.sync_copy(bufs[b],
                                acc_s.at[idx_v.at[ui // nsub, ui % nsub]],
                                add=True)

                @pl.when(u + 2 < nu)
                def _():
                    load(u + 2, b)

            @pl.when(c + 1 < nch)
            def _():
                pltpu.sync_copy(
                    di_hbm.at[pl.ds(wid * nm + (c + 1) * ich, ich)], idx_v)
            return carry

        lax.fori_loop(0, nch, chunk, 0)
        plsc.subcore_barrier()
        pltpu.sync_copy(acc_s.at[pl.ds(r0, rpt)],
                        out_hbm.at[cid, pl.ds(r0, rpt)])

    return kfn(msg, didx3, zeros)


def _tc_tables(h, x128, wa, wb):
    n = h.shape[0]
    f32 = jnp.float32

    def body(h_r, x_r, wa_r, wb_r, st_r, ts_r, td_r):
        hv = h_r[...]
        xv = x_r[...]
        st = st_r[...]
        p = jnp.dot(hv, wa_r[...], preferred_element_type=f32)
        q = jnp.dot(hv, wb_r[...], preferred_element_type=f32)
        ts_r[...] = jnp.dot(p, st, preferred_element_type=f32) + xv
        td_r[...] = jnp.dot(q, st, preferred_element_type=f32) + xv

    st = jnp.eye(HW, RW, dtype=f32)
    return pl.pallas_call(
        body,
        out_shape=(jax.ShapeDtypeStruct((n, RW), f32),
                   jax.ShapeDtypeStruct((n, RW), f32)),
    )(h, x128, wa, wb, st)


def _tc_edge_mlp(gs, gd, eattr, wr, we, b1, w2, b2, cw1, cb1, cw2,
                 e_real, row0, with_coord):
    ep = gs.shape[0]
    blk = 8192
    grid = ep // blk
    f32 = jnp.float32

    def body(*refs):
        if with_coord:
            (gs_r, gd_r, ea_r, sel_r, selt_r, wr_r, we_r, b1_r, w2_r, b2_r,
             cw1_r, cb1_r, cw2_r, out_r) = refs
        else:
            (gs_r, gd_r, ea_r, sel_r, selt_r, wr_r, we_r, b1_r, w2_r, b2_r,
             out_r) = refs
        i = pl.program_id(0)
        gsv = gs_r[...].astype(f32)
        gdv = gd_r[...].astype(f32)
        sel = sel_r[...]
        lane = lax.broadcasted_iota(jnp.int32, (1, RW), 1)
        xmask = jnp.where((lane >= XL) & (lane < XL + 3), 1.0, 0.0).astype(f32)
        xd = (gsv - gdv) * xmask
        radial = jnp.sum(xd * xd, axis=1, keepdims=True)
        pre = (jnp.dot(gsv + gdv, sel, preferred_element_type=f32)
               + radial * wr_r[...]
               + jnp.dot(ea_r[...], we_r[...], preferred_element_type=f32)
               + b1_r[...])
        u = jnp.dot(_silu(pre), w2_r[...], preferred_element_type=f32) + b2_r[...]
        mh = _silu(u)
        rows = row0 + i * blk + lax.broadcasted_iota(jnp.int32, (blk, 1), 0)
        emask = rows < e_real
        selt = selt_r[...]
        out = jnp.dot(mh, selt, preferred_element_type=f32)
        if with_coord:
            coef = jnp.dot(
                _silu(jnp.dot(mh, cw1_r[...], preferred_element_type=f32)
                      + cb1_r[...]),
                cw2_r[...], preferred_element_type=f32)
            inv = 1.0 / (jnp.sqrt(radial) + 1e-30)
            deghot = jnp.where(lane == DEGL, 1.0, 0.0).astype(f32)
            out = out + coef * (xd * inv) + deghot
        out_r[...] = jnp.where(emask, out, 0.0)

    def eb(s):
        return pl.BlockSpec((blk, s), lambda i: (i, 0))

    def wbs(shape):
        return pl.BlockSpec(shape, lambda i: tuple(0 for _ in shape))

    sel = jnp.eye(RW, HW, dtype=f32)
    selt = jnp.eye(HW, RW, dtype=f32)
    in_specs = [eb(RW), eb(RW), eb(eattr.shape[1]), wbs(sel.shape),
                wbs(selt.shape), wbs(wr.shape), wbs(we.shape), wbs(b1.shape),
                wbs(w2.shape), wbs(b2.shape)]
    args = [gs, gd, eattr, sel, selt, wr, we, b1, w2, b2]
    if with_coord:
        in_specs += [wbs(cw1.shape), wbs(cb1.shape), wbs(cw2.shape)]
        args += [cw1, cb1, cw2]

    return pl.pallas_call(
        body,
        grid=(grid,),
        in_specs=in_specs,
        out_specs=eb(RW),
        out_shape=jax.ShapeDtypeStruct((ep, RW), f32),
    )(*args)


def _tc_node1(h0, x128, parts, nw1a, nw1b, nb1, nw2, nb2, wa2, wb2):
    n = h0.shape[0]
    np_ = len(parts)
    f32 = jnp.float32

    def body(*refs):
        (h_r, x_r), ph_rs, (sel_r, selt_r, nw1a_r, nw1b_r, nb1_r,
                            nw2_r, nb2_r, wa2_r, wb2_r,
                            hr_r, ts2_r, td2_r) = (
            refs[:2], refs[2:2 + np_], refs[2 + np_:])
        agg = sum(p[0] + p[1] for p in ph_rs)
        sel = sel_r[...]
        selt = selt_r[...]
        lane = lax.broadcasted_iota(jnp.int32, (1, RW), 1)
        hn = jnp.dot(agg, sel, preferred_element_type=f32)
        degcol = jnp.where(
            lax.broadcasted_iota(jnp.int32, (RW, 1), 0) == DEGL,
            1.0, 0.0).astype(f32)
        deg = jnp.dot(agg, degcol, preferred_element_type=f32)
        xmask = jnp.where((lane >= XL) & (lane < XL + 3), 1.0, 0.0).astype(f32)
        xsum = agg * xmask
        xneigh = xsum / jnp.maximum(deg, 1.0)
        xout = x_r[...] + xneigh
        pre = (jnp.dot(h_r[...], nw1a_r[...], preferred_element_type=f32)
               + jnp.dot(hn, nw1b_r[...], preferred_element_type=f32)
               + nb1_r[...])
        h1 = jnp.dot(_silu(pre), nw2_r[...], preferred_element_type=f32) + nb2_r[...]
        hr = jnp.maximum(h1, 0.0)
        xr = jnp.maximum(xout, 0.0)
        hr_r[...] = hr
        p2 = jnp.dot(hr, wa2_r[...], preferred_element_type=f32)
        q2 = jnp.dot(hr, wb2_r[...], preferred_element_type=f32)
        ts2_r[...] = jnp.dot(p2, selt, preferred_element_type=f32) + xr
        td2_r[...] = jnp.dot(q2, selt, preferred_element_type=f32) + xr

    sel = jnp.eye(RW, HW, dtype=f32)
    selt = jnp.eye(HW, RW, dtype=f32)
    bn = n // 8

    def nb(s):
        return pl.BlockSpec((bn, s), lambda i: (i, 0))

    def pb():
        return pl.BlockSpec((2, bn, RW), lambda i: (0, i, 0))

    def wbs(shape):
        return pl.BlockSpec(shape, lambda i: tuple(0 for _ in shape))

    return pl.pallas_call(
        body,
        grid=(8,),
        in_specs=[nb(h0.shape[1]), nb(RW)] + [pb() for _ in parts]
        + [wbs(sel.shape), wbs(selt.shape), wbs(nw1a.shape),
           wbs(nw1b.shape), wbs(nb1.shape), wbs(nw2.shape), wbs(nb2.shape),
           wbs(wa2.shape), wbs(wb2.shape)],
        out_specs=(nb(HW), nb(RW), nb(RW)),
        out_shape=(jax.ShapeDtypeStruct((n, HW), f32),
                   jax.ShapeDtypeStruct((n, RW), f32),
                   jax.ShapeDtypeStruct((n, RW), f32)),
    )(h0, x128, *parts, sel, selt, nw1a, nw1b, nb1, nw2, nb2, wa2, wb2)


def _tc_node2(hr, parts, nw1a, nw1b, nb1, nw2, nb2, n_real):
    n_pad = hr.shape[0]
    np_ = len(parts)
    f32 = jnp.float32

    bn = n_pad // 8

    def body(*refs):
        (h_r,), ph_rs, (sel_r, nw1a_r, nw1b_r, nb1_r, nw2_r, nb2_r,
                        out_r) = refs[:1], refs[1:1 + np_], refs[1 + np_:]
        i = pl.program_id(0)
        agg = sum(p[0] + p[1] for p in ph_rs)
        hn = jnp.dot(agg, sel_r[...], preferred_element_type=f32)
        pre = (jnp.dot(h_r[...], nw1a_r[...], preferred_element_type=f32)
               + jnp.dot(hn, nw1b_r[...], preferred_element_type=f32)
               + nb1_r[...])
        h2 = jnp.dot(_silu(pre), nw2_r[...], preferred_element_type=f32) + nb2_r[...]
        rows = i * bn + lax.broadcasted_iota(jnp.int32, (bn, 1), 0)
        h2 = jnp.where(rows < n_real, h2, 0.0)
        part = jnp.sum(h2, axis=0, keepdims=True)

        @pl.when(i == 0)
        def _():
            out_r[...] = part

        @pl.when(i > 0)
        def _():
            out_r[...] = out_r[...] + part

    sel = jnp.eye(RW, HW, dtype=f32)

    def nb(s):
        return pl.BlockSpec((bn, s), lambda i: (i, 0))

    def pb():
        return pl.BlockSpec((2, bn, RW), lambda i: (0, i, 0))

    def wbs(shape):
        return pl.BlockSpec(shape, lambda i: tuple(0 for _ in shape))

    return pl.pallas_call(
        body,
        grid=(8,),
        in_specs=[nb(HW)] + [pb() for _ in parts]
        + [wbs(sel.shape), wbs(nw1a.shape), wbs(nw1b.shape),
           wbs(nb1.shape), wbs(nw2.shape), wbs(nb2.shape)],
        out_specs=pl.BlockSpec((1, HW), lambda i: (0, 0)),
        out_shape=jax.ShapeDtypeStruct((1, HW), f32),
    )(hr, *parts, sel, nw1a, nw1b, nb1, nw2, nb2)


def kernel(node_feat, coord_feat, edge_index, edge_attr,
           c1_e_w1, c1_e_b1, c1_e_w2, c1_e_b2, c1_n_w1, c1_n_b1, c1_n_w2,
           c1_n_b2, c1_c_w1, c1_c_b1, c1_c_w2,
           c2_e_w1, c2_e_b1, c2_e_w2, c2_e_b2, c2_n_w1, c2_n_b1, c2_n_w2,
           c2_n_b2, c2_c_w1, c2_c_b1, c2_c_w2):
    n, f_in = node_feat.shape
    e = edge_index.shape[1]
    hw = c1_e_w2.shape[0]
    c = coord_feat.shape[1]

    unit = NW * MACRO
    e_pad = ((e + unit - 1) // unit) * unit
    nunit = 16 * NS
    n_pad = ((n + nunit - 1) // nunit) * nunit

    src = jnp.pad(edge_index[0], (0, e_pad - e))
    dst = jnp.pad(edge_index[1], (0, e_pad - e))
    sidx3 = src.reshape(e_pad // MACRO, MACRO // SUB, SUB)
    didx3 = dst.reshape(e_pad // MACRO, MACRO // SUB, SUB)
    eattr = jnp.pad(edge_attr, ((0, e_pad - e), (0, 0)))
    node_feat = jnp.pad(node_feat, ((0, n_pad - n), (0, 0)))
    x128 = jnp.pad(coord_feat, ((0, n_pad - n), (XL, RW - XL - c)))

    wa1 = c1_e_w1[:f_in]
    wb1 = c1_e_w1[f_in:2 * f_in]
    wr1 = c1_e_w1[2 * f_in:2 * f_in + 1]
    we1 = c1_e_w1[2 * f_in + 1:]
    b1e = c1_e_b1.reshape(1, hw)
    b2e = c1_e_b2.reshape(1, hw)
    cb1 = c1_c_b1.reshape(1, hw)
    nw1a = c1_n_w1[:f_in]
    nw1b = c1_n_w1[f_in:]
    nb1 = c1_n_b1.reshape(1, hw)
    nb2 = c1_n_b2.reshape(1, hw)
    wa2 = c2_e_w1[:hw]
    wb2 = c2_e_w1[hw:2 * hw]
    wr2 = c2_e_w1[2 * hw:2 * hw + 1]
    we2 = c2_e_w1[2 * hw + 1:]
    b1e2 = c2_e_b1.reshape(1, hw)
    b2e2 = c2_e_b2.reshape(1, hw)
    nw1a2 = c2_n_w1[:hw]
    nw1b2 = c2_n_w1[hw:]
    nb12 = c2_n_b1.reshape(1, hw)
    nb22 = c2_n_b2.reshape(1, hw)

    nstream = 2
    nmac = e_pad // MACRO
    halves = []
    for hidx in range(nstream):
        mac0 = hidx * (nmac // nstream)
        row0 = mac0 * MACRO
        halves.append((
            row0,
            sidx3[mac0:mac0 + nmac // nstream],
            didx3[mac0:mac0 + nmac // nstream],
            eattr[row0:row0 + e_pad // nstream],
        ))

    def conv_edges(ts, td, ew, wr, we, b1, b2, cw1, cb1, cw2, with_coord):
        parts = []
        for row0, sa, da, ea in halves:
            gsx, gdx = _sc_gather(ts, td, sa, da)
            msgx = _tc_edge_mlp(gsx, gdx, ea, wr, we, b1, ew, b2,
                                cw1, cb1, cw2, e, row0, with_coord)
            parts.append(_sc_scatter(msgx, da, n_pad))
        return parts

    ts1, td1 = _tc_tables(node_feat, x128, wa1, wb1)
    ph1 = conv_edges(ts1, td1, c1_e_w2, wr1, we1, b1e, b2e,
                     c1_c_w1, cb1, c1_c_w2, True)
    hr, ts2, td2 = _tc_node1(node_feat, x128, ph1, nw1a, nw1b, nb1,
                             c1_n_w2, nb2, wa2, wb2)

    ph2 = conv_edges(ts2, td2, c2_e_w2, wr2, we2, b1e2, b2e2,
                     None, None, None, False)
    return _tc_node2(hr, ph2, nw1a2, nw1b2, nb12, c2_n_w2, nb22, n)
